# Initial kernel scaffold; baseline (speedup 1.0000x reference)
import jax
import jax.numpy as jnp
from jax.experimental import pallas as pl


def kernel(confidence, predicted_locations, gt_labels, gt_locations):
    raise NotImplementedError("write your pallas kernel here")


# speedup vs baseline: 1.3569x; 1.3569x over previous
import functools

import jax
import jax.numpy as jnp
from jax.experimental import pallas as pl

_NEG_POS_RATIO = 3.0


def _stage_a(conf_ref, lab_ref, ploc_ref, gloc_ref,
             nv_ref, npos_ref, posce_ref, sl1_ref):
    x = conf_ref[0]
    P, C = x.shape
    m = jnp.max(x, axis=1, keepdims=True)
    e = jnp.exp(x - m)
    lse = m + jnp.log(jnp.sum(e, axis=1, keepdims=True))

    lab = lab_ref[0]
    pos = lab > 0.0

    cls = jax.lax.broadcasted_iota(jnp.int32, (P, C), 1)
    labi = lab.astype(jnp.int32)
    clabel = jnp.sum(jnp.where(cls == labi, x, 0.0), axis=1, keepdims=True)
    ce = lse - clabel
    mining = lse - x[:, 0:1]

    nv_ref[0] = jnp.where(pos, -1.0, mining)

    posf = jnp.where(pos, 1.0, 0.0)
    npos_ref[...] = jnp.sum(posf).reshape(1, 1, 1)
    posce_ref[...] = jnp.sum(jnp.where(pos, ce, 0.0)).reshape(1, 1, 1)

    d = ploc_ref[0] - gloc_ref[0]
    ad = jnp.abs(d)
    sl1 = jnp.where(ad < 1.0, 0.5 * d * d, ad - 0.5)
    sl1_ref[...] = jnp.sum(jnp.where(pos, sl1, 0.0)).reshape(1, 1, 1)


def _stage_b(nv_ref, npos_ref, posce_ref, sl1_ref, out0_ref, out1_ref, *, P):
    nv = nv_ref[...]
    npos = npos_ref[...]
    k = jnp.minimum(_NEG_POS_RATIO * npos, float(P) - npos)
    ki = k.astype(jnp.int32)

    iv = jax.lax.bitcast_convert_type(nv, jnp.int32)
    t = jnp.zeros(npos.shape, jnp.int32)
    for bit in range(30, -1, -1):
        t2 = t | (1 << bit)
        cnt = jnp.sum((iv >= t2).astype(jnp.int32), axis=1, keepdims=True)
        t = jnp.where(cnt >= ki, t2, t)
    vk = jax.lax.bitcast_convert_type(t, jnp.float32)
    gt = iv > t
    cnt_gt = jnp.sum(gt.astype(jnp.float32), axis=1, keepdims=True)
    sum_gt = jnp.sum(jnp.where(gt, nv, 0.0), axis=1, keepdims=True)
    topk = jnp.where(ki > 0, sum_gt + (k - cnt_gt) * vk, 0.0)

    npos_tot = jnp.sum(npos)
    out0_ref[...] = (jnp.sum(sl1_ref[...]) / npos_tot).reshape(1, 1)
    out1_ref[...] = ((jnp.sum(posce_ref[...]) + jnp.sum(topk))
                     / npos_tot).reshape(1, 1)


@jax.jit
def kernel(confidence, predicted_locations, gt_labels, gt_locations):
    B, P, C = confidence.shape
    labels_f = gt_labels.astype(jnp.float32).reshape(B, P, 1)

    nv, npos, posce, sl1 = pl.pallas_call(
        _stage_a,
        grid=(B,),
        in_specs=[
            pl.BlockSpec((1, P, C), lambda b: (b, 0, 0)),
            pl.BlockSpec((1, P, 1), lambda b: (b, 0, 0)),
            pl.BlockSpec((1, P, 4), lambda b: (b, 0, 0)),
            pl.BlockSpec((1, P, 4), lambda b: (b, 0, 0)),
        ],
        out_specs=[
            pl.BlockSpec((1, P, 1), lambda b: (b, 0, 0)),
            pl.BlockSpec((1, 1, 1), lambda b: (b, 0, 0)),
            pl.BlockSpec((1, 1, 1), lambda b: (b, 0, 0)),
            pl.BlockSpec((1, 1, 1), lambda b: (b, 0, 0)),
        ],
        out_shape=[
            jax.ShapeDtypeStruct((B, P, 1), jnp.float32),
            jax.ShapeDtypeStruct((B, 1, 1), jnp.float32),
            jax.ShapeDtypeStruct((B, 1, 1), jnp.float32),
            jax.ShapeDtypeStruct((B, 1, 1), jnp.float32),
        ],
    )(confidence, labels_f, predicted_locations, gt_locations)

    out0, out1 = pl.pallas_call(
        functools.partial(_stage_b, P=P),
        out_shape=[
            jax.ShapeDtypeStruct((1, 1), jnp.float32),
            jax.ShapeDtypeStruct((1, 1), jnp.float32),
        ],
    )(nv.reshape(B, P), npos.reshape(B, 1), posce.reshape(B, 1),
      sl1.reshape(B, 1))

    return (out0[0, 0], out1[0, 0])


# speedup vs baseline: 4.9378x; 3.6390x over previous
import functools

import jax
import jax.numpy as jnp
from jax.experimental import pallas as pl

_NEG_POS_RATIO = 3.0


def _stage_a(conf_ref, lab_ref, ploc_ref, gloc_ref,
             nv_ref, npos_ref, posce_ref, sl1_ref):
    x = conf_ref[0]
    s = jnp.sum(jnp.exp(x), axis=0, keepdims=True)
    lse = jnp.log(s)

    lab = lab_ref[0]
    pos = lab > 0.0

    cls = jax.lax.broadcasted_iota(jnp.int32, x.shape, 0)
    labi = lab.astype(jnp.int32)
    clabel = jnp.sum(jnp.where(cls == labi, x, 0.0), axis=0, keepdims=True)
    ce = lse - clabel
    mining = lse - x[0:1, :]

    nv_ref[0] = jnp.where(pos, -1.0, mining)

    npos_ref[...] = jnp.sum(jnp.where(pos, 1.0, 0.0)).reshape(1, 1, 1)
    posce_ref[...] = jnp.sum(jnp.where(pos, ce, 0.0)).reshape(1, 1, 1)

    d = ploc_ref[0] - gloc_ref[0]
    ad = jnp.abs(d)
    sl1 = jnp.where(ad < 1.0, 0.5 * d * d, ad - 0.5)
    sl1_ref[...] = jnp.sum(jnp.where(pos, sl1, 0.0)).reshape(1, 1, 1)


def _stage_b(nv_ref, npos_ref, posce_ref, sl1_ref, out0_ref, out1_ref, *, P):
    nv = nv_ref[...]
    npos = npos_ref[...]
    k = jnp.minimum(_NEG_POS_RATIO * npos, float(P) - npos)
    ki = k.astype(jnp.int32)

    iv = jax.lax.bitcast_convert_type(nv, jnp.int32)
    t = jnp.zeros(npos.shape, jnp.int32)
    for bit in range(30, -1, -1):
        t2 = t | (1 << bit)
        cnt = jnp.sum((iv >= t2).astype(jnp.int32), axis=1, keepdims=True)
        t = jnp.where(cnt >= ki, t2, t)
    vk = jax.lax.bitcast_convert_type(t, jnp.float32)
    gt = iv > t
    cnt_gt = jnp.sum(gt.astype(jnp.float32), axis=1, keepdims=True)
    sum_gt = jnp.sum(jnp.where(gt, nv, 0.0), axis=1, keepdims=True)
    topk = jnp.where(ki > 0, sum_gt + (k - cnt_gt) * vk, 0.0)

    npos_tot = jnp.sum(npos)
    out0_ref[...] = (jnp.sum(sl1_ref[...]) / npos_tot).reshape(1, 1)
    out1_ref[...] = ((jnp.sum(posce_ref[...]) + jnp.sum(topk))
                     / npos_tot).reshape(1, 1)


@jax.jit
def kernel(confidence, predicted_locations, gt_labels, gt_locations):
    B, P, C = confidence.shape
    conf_t = jnp.swapaxes(confidence, 1, 2)
    ploc_t = jnp.swapaxes(predicted_locations, 1, 2)
    gloc_t = jnp.swapaxes(gt_locations, 1, 2)
    labels_f = gt_labels.astype(jnp.float32).reshape(B, 1, P)

    nv, npos, posce, sl1 = pl.pallas_call(
        _stage_a,
        grid=(B,),
        in_specs=[
            pl.BlockSpec((1, C, P), lambda b: (b, 0, 0)),
            pl.BlockSpec((1, 1, P), lambda b: (b, 0, 0)),
            pl.BlockSpec((1, 4, P), lambda b: (b, 0, 0)),
            pl.BlockSpec((1, 4, P), lambda b: (b, 0, 0)),
        ],
        out_specs=[
            pl.BlockSpec((1, 1, P), lambda b: (b, 0, 0)),
            pl.BlockSpec((1, 1, 1), lambda b: (b, 0, 0)),
            pl.BlockSpec((1, 1, 1), lambda b: (b, 0, 0)),
            pl.BlockSpec((1, 1, 1), lambda b: (b, 0, 0)),
        ],
        out_shape=[
            jax.ShapeDtypeStruct((B, 1, P), jnp.float32),
            jax.ShapeDtypeStruct((B, 1, 1), jnp.float32),
            jax.ShapeDtypeStruct((B, 1, 1), jnp.float32),
            jax.ShapeDtypeStruct((B, 1, 1), jnp.float32),
        ],
    )(conf_t, labels_f, ploc_t, gloc_t)

    out0, out1 = pl.pallas_call(
        functools.partial(_stage_b, P=P),
        out_shape=[
            jax.ShapeDtypeStruct((1, 1), jnp.float32),
            jax.ShapeDtypeStruct((1, 1), jnp.float32),
        ],
    )(nv.reshape(B, P), npos.reshape(B, 1), posce.reshape(B, 1),
      sl1.reshape(B, 1))

    return (out0[0, 0], out1[0, 0])
